# Initial kernel scaffold; baseline (speedup 1.0000x reference)
import jax
import jax.numpy as jnp
from jax.experimental import pallas as pl


def kernel(new_node, rbf, edge_f, edge_index, W1, b1, W2, b2, W3, b3):
    raise NotImplementedError("write your pallas kernel here")


# speedup vs baseline: 1.7933x; 1.7933x over previous
import functools

import jax
import jax.numpy as jnp
from jax import lax
from jax.experimental import pallas as pl
from jax.experimental.pallas import tpu as pltpu
from jax.experimental.pallas import tpu_sc as plsc

N_NODES = 50000
N_EDGES = 800000
DIM = 64

NC = 2
NS = 16
NW = NC * NS

G_IDXW = 125
G_ROWS_PER_W = (N_EDGES // G_IDXW) // NW
G_CHUNK_ROWS = 8
G_NCH = G_ROWS_PER_W // G_CHUNK_ROWS
G_CHUNK_E = G_CHUNK_ROWS * G_IDXW

_gather_mesh = plsc.VectorSubcoreMesh(core_axis_name="c", subcore_axis_name="s")


@functools.partial(
    pl.kernel,
    mesh=_gather_mesh,
    compiler_params=pltpu.CompilerParams(use_tc_tiling_on_sc=False),
    out_type=jax.ShapeDtypeStruct((N_EDGES, DIM), jnp.float32),
    scratch_types=[
        pltpu.VMEM((G_CHUNK_ROWS, G_IDXW), jnp.int32),
        pltpu.VMEM((G_CHUNK_E, DIM), jnp.float32),
        pltpu.SemaphoreType.DMA,
    ],
)
def _sc_gather(node_hbm, src2d_hbm, g_hbm, idx_v, rows_v, sem):
    c = lax.axis_index("c")
    s = lax.axis_index("s")
    wid = s * NC + c
    row0 = wid * G_ROWS_PER_W

    def body(i, carry):
        rb = row0 + i * G_CHUNK_ROWS
        pltpu.sync_copy(src2d_hbm.at[pl.ds(rb, G_CHUNK_ROWS)], idx_v)
        cps = [
            pltpu.async_copy(
                node_hbm.at[idx_v.at[j]],
                rows_v.at[pl.ds(j * G_IDXW, G_IDXW)],
                sem,
            )
            for j in range(G_CHUNK_ROWS)
        ]
        for cp in cps:
            cp.wait()
        pltpu.sync_copy(rows_v, g_hbm.at[pl.ds(rb * G_IDXW, G_CHUNK_E)])
        return carry

    lax.fori_loop(0, G_NCH, body, None)


TC_BLK = 5000


def _tc_body(g_ref, rbf_ref, ef_ref, w1_ref, b1_ref, w2_ref, b2_ref,
             w3_ref, b3_ref, o_ref):
    h = jnp.dot(rbf_ref[...], w1_ref[...], preferred_element_type=jnp.float32)
    h = h + b1_ref[...]
    z = 0.5 * h
    sp = 2.0 * jnp.log1p(jnp.exp(jnp.minimum(z, 14.0)))
    h = jnp.where(z > 14.0, h, sp)
    h = jnp.dot(h, w2_ref[...], preferred_element_type=jnp.float32)
    h = h + b2_ref[...]
    e2 = jnp.dot(ef_ref[...], w3_ref[...], preferred_element_type=jnp.float32)
    e2 = e2 + b3_ref[...]
    o_ref[...] = g_ref[...] * h + e2


def _tc_dense(g, rbf, edge_f, W1, b1, W2, b2, W3, b3):
    blk = pl.BlockSpec((TC_BLK, DIM), lambda i: (i, 0))
    wspec = pl.BlockSpec((DIM, DIM), lambda i: (0, 0))
    bspec = pl.BlockSpec((1, DIM), lambda i: (0, 0))
    return pl.pallas_call(
        _tc_body,
        grid=(N_EDGES // TC_BLK,),
        in_specs=[blk, blk, blk, wspec, bspec, wspec, bspec, wspec, bspec],
        out_specs=blk,
        out_shape=jax.ShapeDtypeStruct((N_EDGES, DIM), jnp.float32),
    )(g, rbf, edge_f, W1, b1.reshape(1, DIM), W2, b2.reshape(1, DIM),
      W3, b3.reshape(1, DIM))


HALF = N_NODES // NC
ACC_ROWS = HALF + 8
S_BW = 80
S_KB = 5
S_CHUNK = S_KB * S_BW
S_PER_TILE = N_EDGES // NS
S_NCH = S_PER_TILE // S_CHUNK
S_WR = HALF // NS
S_WR_TAIL = HALF - S_WR * NS

_scatter_mesh = plsc.VectorSubcoreMesh(core_axis_name="c", subcore_axis_name="s")


@functools.partial(
    pl.kernel,
    mesh=_scatter_mesh,
    compiler_params=pltpu.CompilerParams(use_tc_tiling_on_sc=False),
    out_type=jax.ShapeDtypeStruct((N_NODES, DIM), jnp.float32),
    scratch_types=[
        pltpu.VMEM((S_KB, S_BW), jnp.int32),
        pltpu.VMEM((S_KB, S_BW), jnp.int32),
        pltpu.VMEM((S_CHUNK, DIM), jnp.float32),
        pltpu.VMEM_SHARED((ACC_ROWS, DIM), jnp.float32),
    ],
)
def _sc_scatter(msg_hbm, dst2d_hbm, z_hbm, out_hbm, dbuf, lbuf, mbuf, acc):
    c = lax.axis_index("c")
    s = lax.axis_index("s")
    base = c * HALF

    @pl.when(s == 0)
    def _():
        pltpu.sync_copy(z_hbm, acc)

    plsc.subcore_barrier()

    row0 = s * (S_PER_TILE // S_BW)

    def body(i, carry):
        rb = row0 + i * S_KB
        pltpu.sync_copy(dst2d_hbm.at[pl.ds(rb, S_KB)], dbuf)
        for j in range(S_KB):
            for t in range(S_BW // 16):
                d = dbuf[j, pl.ds(t * 16, 16)]
                ok = (d >= base) & (d < base + HALF)
                lbuf[j, pl.ds(t * 16, 16)] = jnp.where(ok, d - base, HALF)
        pltpu.sync_copy(msg_hbm.at[pl.ds(rb * S_BW, S_CHUNK)], mbuf)
        for j in range(S_KB):
            pltpu.sync_copy(
                mbuf.at[pl.ds(j * S_BW, S_BW)],
                acc.at[lbuf.at[j]],
                add=True,
            )
        return carry

    lax.fori_loop(0, S_NCH, body, None)
    plsc.subcore_barrier()

    pltpu.sync_copy(
        acc.at[pl.ds(s * S_WR, S_WR)],
        out_hbm.at[pl.ds(base + s * S_WR, S_WR)],
    )

    @pl.when(s == 0)
    def _():
        pltpu.sync_copy(
            acc.at[pl.ds(NS * S_WR, S_WR_TAIL)],
            out_hbm.at[pl.ds(base + NS * S_WR, S_WR_TAIL)],
        )


def kernel(new_node, rbf, edge_f, edge_index, W1, b1, W2, b2, W3, b3):
    src = edge_index[0].astype(jnp.int32)
    dst = edge_index[1].astype(jnp.int32)
    src2d = src.reshape(N_EDGES // G_IDXW, G_IDXW)
    dst2d = dst.reshape(N_EDGES // S_BW, S_BW)
    g = _sc_gather(new_node, src2d)
    msg = _tc_dense(g, rbf, edge_f, W1, b1, W2, b2, W3, b3)
    z = jnp.zeros((ACC_ROWS, DIM), jnp.float32)
    return _sc_scatter(msg, dst2d, z)


# speedup vs baseline: 2.3788x; 1.3265x over previous
import functools

import jax
import jax.numpy as jnp
from jax import lax
from jax.experimental import pallas as pl
from jax.experimental.pallas import tpu as pltpu
from jax.experimental.pallas import tpu_sc as plsc

N_NODES = 50000
N_EDGES = 800000
DIM = 64

NC = 2
NS = 16
NW = NC * NS

G_IDXW = 125
G_ROWS_PER_W = (N_EDGES // G_IDXW) // NW
G_CHUNK_ROWS = 8
G_NCH = G_ROWS_PER_W // G_CHUNK_ROWS
G_CHUNK_E = G_CHUNK_ROWS * G_IDXW

_gather_mesh = plsc.VectorSubcoreMesh(core_axis_name="c", subcore_axis_name="s")


@functools.partial(
    pl.kernel,
    mesh=_gather_mesh,
    compiler_params=pltpu.CompilerParams(use_tc_tiling_on_sc=False),
    out_type=jax.ShapeDtypeStruct((N_EDGES, DIM), jnp.float32),
    scratch_types=[
        pltpu.VMEM((G_CHUNK_ROWS, G_IDXW), jnp.int32),
        pltpu.VMEM((G_CHUNK_E, DIM), jnp.float32),
        pltpu.SemaphoreType.DMA,
    ],
)
def _sc_gather(node_hbm, src2d_hbm, g_hbm, idx_v, rows_v, sem):
    c = lax.axis_index("c")
    s = lax.axis_index("s")
    wid = s * NC + c
    row0 = wid * G_ROWS_PER_W

    def body(i, carry):
        rb = row0 + i * G_CHUNK_ROWS
        pltpu.sync_copy(src2d_hbm.at[pl.ds(rb, G_CHUNK_ROWS)], idx_v)
        cps = [
            pltpu.async_copy(
                node_hbm.at[idx_v.at[j]],
                rows_v.at[pl.ds(j * G_IDXW, G_IDXW)],
                sem,
            )
            for j in range(G_CHUNK_ROWS)
        ]
        for cp in cps:
            cp.wait()
        pltpu.sync_copy(rows_v, g_hbm.at[pl.ds(rb * G_IDXW, G_CHUNK_E)])
        return carry

    lax.fori_loop(0, G_NCH, body, None)


TC_BLK = 6400


def _tc_body(g_ref, rbft_ref, eft_ref, w1t_ref, b1_ref, w2_ref, b2_ref,
             w3_ref, b3_ref, o_ref):
    ht = jnp.dot(w1t_ref[...], rbft_ref[...], preferred_element_type=jnp.float32)
    ht = ht + b1_ref[...]
    z = 0.5 * ht
    sp = 2.0 * jnp.log1p(jnp.exp(jnp.minimum(z, 14.0)))
    ht = jnp.where(z > 14.0, ht, sp)
    h = lax.dot_general(ht, w2_ref[...], (((0,), (0,)), ((), ())),
                        preferred_element_type=jnp.float32)
    h = h + b2_ref[...]
    e2 = lax.dot_general(eft_ref[...], w3_ref[...], (((0,), (0,)), ((), ())),
                         preferred_element_type=jnp.float32)
    e2 = e2 + b3_ref[...]
    o_ref[...] = g_ref[...] * h + e2


def _tc_dense(g, rbf, edge_f, W1, b1, W2, b2, W3, b3):
    blk = pl.BlockSpec((TC_BLK, DIM), lambda i: (i, 0))
    blk_t = pl.BlockSpec((DIM, TC_BLK), lambda i: (0, i))
    wspec = pl.BlockSpec((DIM, DIM), lambda i: (0, 0))
    brow = pl.BlockSpec((1, DIM), lambda i: (0, 0))
    bcol = pl.BlockSpec((DIM, 1), lambda i: (0, 0))
    return pl.pallas_call(
        _tc_body,
        grid=(N_EDGES // TC_BLK,),
        in_specs=[blk, blk_t, blk_t, wspec, bcol, wspec, brow, wspec, brow],
        out_specs=blk,
        out_shape=jax.ShapeDtypeStruct((N_EDGES, DIM), jnp.float32),
    )(g, rbf.T, edge_f.T, W1.T, b1.reshape(DIM, 1), W2, b2.reshape(1, DIM),
      W3, b3.reshape(1, DIM))


HALF = N_NODES // NC
ACC_ROWS = HALF + 8
S_BW = 80
S_KB = 5
S_CHUNK = S_KB * S_BW
S_PER_TILE = N_EDGES // NS
S_NCH = S_PER_TILE // S_CHUNK
S_WR = HALF // NS
S_WR_TAIL = HALF - S_WR * NS

_scatter_mesh = plsc.VectorSubcoreMesh(core_axis_name="c", subcore_axis_name="s")


@functools.partial(
    pl.kernel,
    mesh=_scatter_mesh,
    compiler_params=pltpu.CompilerParams(use_tc_tiling_on_sc=False),
    out_type=jax.ShapeDtypeStruct((N_NODES, DIM), jnp.float32),
    scratch_types=[
        pltpu.VMEM((S_KB, S_BW), jnp.int32),
        pltpu.VMEM((S_KB, S_BW), jnp.int32),
        pltpu.VMEM((S_CHUNK, DIM), jnp.float32),
        pltpu.VMEM_SHARED((ACC_ROWS, DIM), jnp.float32),
    ],
)
def _sc_scatter(msg_hbm, dst2d_hbm, z_hbm, out_hbm, dbuf, lbuf, mbuf, acc):
    c = lax.axis_index("c")
    s = lax.axis_index("s")
    base = c * HALF

    @pl.when(s == 0)
    def _():
        pltpu.sync_copy(z_hbm, acc)

    plsc.subcore_barrier()

    row0 = s * (S_PER_TILE // S_BW)

    def body(i, carry):
        rb = row0 + i * S_KB
        pltpu.sync_copy(dst2d_hbm.at[pl.ds(rb, S_KB)], dbuf)
        for j in range(S_KB):
            for t in range(S_BW // 16):
                d = dbuf[j, pl.ds(t * 16, 16)]
                ok = (d >= base) & (d < base + HALF)
                lbuf[j, pl.ds(t * 16, 16)] = jnp.where(ok, d - base, HALF)
        pltpu.sync_copy(msg_hbm.at[pl.ds(rb * S_BW, S_CHUNK)], mbuf)
        for j in range(S_KB):
            pltpu.sync_copy(
                mbuf.at[pl.ds(j * S_BW, S_BW)],
                acc.at[lbuf.at[j]],
                add=True,
            )
        return carry

    lax.fori_loop(0, S_NCH, body, None)
    plsc.subcore_barrier()

    pltpu.sync_copy(
        acc.at[pl.ds(s * S_WR, S_WR)],
        out_hbm.at[pl.ds(base + s * S_WR, S_WR)],
    )

    @pl.when(s == 0)
    def _():
        pltpu.sync_copy(
            acc.at[pl.ds(NS * S_WR, S_WR_TAIL)],
            out_hbm.at[pl.ds(base + NS * S_WR, S_WR_TAIL)],
        )


def kernel(new_node, rbf, edge_f, edge_index, W1, b1, W2, b2, W3, b3):
    src = edge_index[0].astype(jnp.int32)
    dst = edge_index[1].astype(jnp.int32)
    src2d = src.reshape(N_EDGES // G_IDXW, G_IDXW)
    dst2d = dst.reshape(N_EDGES // S_BW, S_BW)
    g = _sc_gather(new_node, src2d)
    msg = _tc_dense(g, rbf, edge_f, W1, b1, W2, b2, W3, b3)
    z = jnp.zeros((ACC_ROWS, DIM), jnp.float32)
    return _sc_scatter(msg, dst2d, z)


# speedup vs baseline: 2.4603x; 1.0343x over previous
import functools

import jax
import jax.numpy as jnp
from jax import lax
from jax.experimental import pallas as pl
from jax.experimental.pallas import tpu as pltpu
from jax.experimental.pallas import tpu_sc as plsc

N_NODES = 50000
N_EDGES = 800000
DIM = 64

NC = 2
NS = 16
NW = NC * NS

G_IDXW = 125
G_ROWS_PER_W = (N_EDGES // G_IDXW) // NW
G_CHUNK_ROWS = 8
G_NCH = G_ROWS_PER_W // G_CHUNK_ROWS
G_CHUNK_E = G_CHUNK_ROWS * G_IDXW

_gather_mesh = plsc.VectorSubcoreMesh(core_axis_name="c", subcore_axis_name="s")


@functools.partial(
    pl.kernel,
    mesh=_gather_mesh,
    compiler_params=pltpu.CompilerParams(use_tc_tiling_on_sc=False),
    out_type=jax.ShapeDtypeStruct((N_EDGES, DIM), jnp.bfloat16),
    scratch_types=[
        pltpu.VMEM((G_CHUNK_ROWS, G_IDXW), jnp.int32),
        pltpu.VMEM((G_CHUNK_E, DIM), jnp.bfloat16),
        pltpu.SemaphoreType.DMA,
    ],
)
def _sc_gather(node_hbm, src2d_hbm, g_hbm, idx_v, rows_v, sem):
    c = lax.axis_index("c")
    s = lax.axis_index("s")
    wid = s * NC + c
    row0 = wid * G_ROWS_PER_W

    def body(i, carry):
        rb = row0 + i * G_CHUNK_ROWS
        pltpu.sync_copy(src2d_hbm.at[pl.ds(rb, G_CHUNK_ROWS)], idx_v)
        cps = [
            pltpu.async_copy(
                node_hbm.at[idx_v.at[j]],
                rows_v.at[pl.ds(j * G_IDXW, G_IDXW)],
                sem,
            )
            for j in range(G_CHUNK_ROWS)
        ]
        for cp in cps:
            cp.wait()
        pltpu.sync_copy(rows_v, g_hbm.at[pl.ds(rb * G_IDXW, G_CHUNK_E)])
        return carry

    lax.fori_loop(0, G_NCH, body, None)


TC_BLK = 6400


def _tc_body(g_ref, rbft_ref, eft_ref, w1t_ref, b1_ref, w2_ref, b2_ref,
             w3_ref, b3_ref, o_ref):
    ht = jnp.dot(w1t_ref[...], rbft_ref[...], preferred_element_type=jnp.float32)
    ht = ht + b1_ref[...]
    z = 0.5 * ht
    sp = 2.0 * jnp.log1p(jnp.exp(jnp.minimum(z, 14.0)))
    ht = jnp.where(z > 14.0, ht, sp)
    h = lax.dot_general(ht, w2_ref[...], (((0,), (0,)), ((), ())),
                        preferred_element_type=jnp.float32)
    h = h + b2_ref[...]
    e2 = lax.dot_general(eft_ref[...], w3_ref[...], (((0,), (0,)), ((), ())),
                         preferred_element_type=jnp.float32)
    e2 = e2 + b3_ref[...]
    o_ref[...] = g_ref[...].astype(jnp.float32) * h + e2


def _tc_dense(g, rbf, edge_f, W1, b1, W2, b2, W3, b3):
    blk = pl.BlockSpec((TC_BLK, DIM), lambda i: (i, 0))
    blk_t = pl.BlockSpec((DIM, TC_BLK), lambda i: (0, i))
    wspec = pl.BlockSpec((DIM, DIM), lambda i: (0, 0))
    brow = pl.BlockSpec((1, DIM), lambda i: (0, 0))
    bcol = pl.BlockSpec((DIM, 1), lambda i: (0, 0))
    return pl.pallas_call(
        _tc_body,
        grid=(N_EDGES // TC_BLK,),
        in_specs=[blk, blk_t, blk_t, wspec, bcol, wspec, brow, wspec, brow],
        out_specs=blk,
        out_shape=jax.ShapeDtypeStruct((N_EDGES, DIM), jnp.float32),
    )(g, rbf.T, edge_f.T, W1.T, b1.reshape(DIM, 1), W2, b2.reshape(1, DIM),
      W3, b3.reshape(1, DIM))


HALF = N_NODES // NC
N_TRASH = 16
ACC_ROWS = HALF + N_TRASH + 8
S_BW = 80
S_KB = 5
S_CHUNK = S_KB * S_BW
S_PER_TILE = N_EDGES // NS
S_ROWS_PER_TILE = S_PER_TILE // S_BW
S_NCH = S_PER_TILE // S_CHUNK
S_WR = HALF // NS
S_WR_TAIL = HALF - S_WR * NS

_scatter_mesh = plsc.VectorSubcoreMesh(core_axis_name="c", subcore_axis_name="s")


@functools.partial(
    pl.kernel,
    mesh=_scatter_mesh,
    compiler_params=pltpu.CompilerParams(use_tc_tiling_on_sc=False),
    out_type=jax.ShapeDtypeStruct((N_NODES, DIM), jnp.float32),
    scratch_types=[
        pltpu.VMEM((S_KB, S_BW), jnp.int32),
        [pltpu.VMEM((S_BW, DIM), jnp.float32) for _ in range(S_KB)],
        pltpu.VMEM_SHARED((ACC_ROWS, DIM), jnp.float32),
        pltpu.SemaphoreType.DMA,
        pltpu.SemaphoreType.DMA,
    ],
)
def _sc_scatter(msg_hbm, dst2d_hbm, z_hbm, out_hbm, dbuf, mbufs, acc,
                sem_in, sem_sc):
    c = lax.axis_index("c")
    s = lax.axis_index("s")
    base = c * HALF

    @pl.when(s == 0)
    def _():
        pltpu.sync_copy(z_hbm, acc)

    plsc.subcore_barrier()

    row0 = s * S_ROWS_PER_TILE

    def group(g, carry):
        rb = row0 + g * S_KB
        pltpu.sync_copy(dst2d_hbm.at[pl.ds(rb, S_KB)], dbuf)
        for j in range(S_KB):
            for t in range(S_BW // 16):
                d = dbuf[j, pl.ds(t * 16, 16)]
                ok = (d >= base) & (d < base + HALF)
                trash = HALF + (d & (N_TRASH - 1))
                dbuf[j, pl.ds(t * 16, 16)] = jnp.where(ok, d - base, trash)
        loads = [
            pltpu.async_copy(
                msg_hbm.at[pl.ds((rb + j) * S_BW, S_BW)], mbufs[j], sem_in
            )
            for j in range(S_KB)
        ]
        stores = []
        for j in range(S_KB):
            loads[j].wait()
            stores.append(
                pltpu.async_copy(
                    mbufs[j], acc.at[dbuf.at[j]], sem_sc, add=True
                )
            )
        for cp in stores:
            cp.wait()
        return carry

    lax.fori_loop(0, S_NCH, group, None)
    plsc.subcore_barrier()

    pltpu.sync_copy(
        acc.at[pl.ds(s * S_WR, S_WR)],
        out_hbm.at[pl.ds(base + s * S_WR, S_WR)],
    )

    @pl.when(s == 0)
    def _():
        pltpu.sync_copy(
            acc.at[pl.ds(NS * S_WR, S_WR_TAIL)],
            out_hbm.at[pl.ds(base + NS * S_WR, S_WR_TAIL)],
        )


def kernel(new_node, rbf, edge_f, edge_index, W1, b1, W2, b2, W3, b3):
    src = edge_index[0].astype(jnp.int32)
    dst = edge_index[1].astype(jnp.int32)
    src2d = src.reshape(N_EDGES // G_IDXW, G_IDXW)
    dst2d = dst.reshape(N_EDGES // S_BW, S_BW)
    g = _sc_gather(new_node.astype(jnp.bfloat16), src2d)
    msg = _tc_dense(g, rbf, edge_f, W1, b1, W2, b2, W3, b3)
    z = jnp.zeros((ACC_ROWS, DIM), jnp.float32)
    return _sc_scatter(msg, dst2d, z)


# speedup vs baseline: 3.1054x; 1.2622x over previous
import functools

import jax
import jax.numpy as jnp
from jax import lax
from jax.experimental import pallas as pl
from jax.experimental.pallas import tpu as pltpu
from jax.experimental.pallas import tpu_sc as plsc

N_NODES = 50000
N_EDGES = 800000
E_HALF = N_EDGES // 2
DIM = 64

NC = 2
NS = 16
NW = NC * NS

G_IDXW = 125
G_ROWS_PER_W = (N_EDGES // G_IDXW) // NW
G_CHUNK_ROWS = 8
G_NCH = G_ROWS_PER_W // G_CHUNK_ROWS
G_CHUNK_E = G_CHUNK_ROWS * G_IDXW
G_EDGES_PER_W = N_EDGES // NW

_gather_mesh = plsc.VectorSubcoreMesh(core_axis_name="c", subcore_axis_name="s")


@functools.partial(
    pl.kernel,
    mesh=_gather_mesh,
    compiler_params=pltpu.CompilerParams(use_tc_tiling_on_sc=False),
    out_type=jax.ShapeDtypeStruct((E_HALF, 2 * DIM), jnp.bfloat16),
    scratch_types=[
        pltpu.VMEM((G_CHUNK_ROWS, G_IDXW), jnp.int32),
        pltpu.VMEM((G_CHUNK_E, DIM), jnp.bfloat16),
        pltpu.SemaphoreType.DMA,
    ],
)
def _sc_gather(node_hbm, src2d_hbm, g_hbm, idx_v, rows_v, sem):
    c = lax.axis_index("c")
    s = lax.axis_index("s")
    wid = s * NC + c
    row0 = wid * G_ROWS_PER_W
    half_row0 = (wid % 16) * G_EDGES_PER_W

    def body(i, carry):
        rb = row0 + i * G_CHUNK_ROWS
        pltpu.sync_copy(src2d_hbm.at[pl.ds(rb, G_CHUNK_ROWS)], idx_v)
        cps = [
            pltpu.async_copy(
                node_hbm.at[idx_v.at[j]],
                rows_v.at[pl.ds(j * G_IDXW, G_IDXW)],
                sem,
            )
            for j in range(G_CHUNK_ROWS)
        ]
        for cp in cps:
            cp.wait()
        orow = half_row0 + i * G_CHUNK_E

        @pl.when(wid < 16)
        def _():
            pltpu.sync_copy(
                rows_v, g_hbm.at[pl.ds(orow, G_CHUNK_E), pl.ds(0, DIM)]
            )

        @pl.when(wid >= 16)
        def _():
            pltpu.sync_copy(
                rows_v, g_hbm.at[pl.ds(orow, G_CHUNK_E), pl.ds(DIM, DIM)]
            )

        return carry

    lax.fori_loop(0, G_NCH, body, None)


TC_BLK = 6400
TC_BLK2 = TC_BLK // 2
TC_NBLK = N_EDGES // TC_BLK


def _tc_body(g_ref, ra_ref, rb_ref, ea_ref, eb_ref, w1t_ref, b1_ref,
             w2_ref, b2_ref, w3_ref, b3_ref, o_ref):
    def mlp_rows(rt):
        ht = jnp.dot(w1t_ref[...], rt, preferred_element_type=jnp.float32)
        ht = ht + b1_ref[...]
        z = 0.5 * ht
        sp = 2.0 * jnp.log1p(jnp.exp(jnp.minimum(z, 14.0)))
        ht = jnp.where(z > 14.0, ht, sp)
        h = lax.dot_general(ht, w2_ref[...], (((0,), (0,)), ((), ())),
                            preferred_element_type=jnp.float32)
        return h + b2_ref[...]

    def lin_rows(et):
        e2 = lax.dot_general(et, w3_ref[...], (((0,), (0,)), ((), ())),
                             preferred_element_type=jnp.float32)
        return e2 + b3_ref[...]

    g = g_ref[...].astype(jnp.float32)
    o_ref[:, 0:DIM] = g[:, 0:DIM] * mlp_rows(ra_ref[...]) + lin_rows(ea_ref[...])
    o_ref[:, DIM:2 * DIM] = (
        g[:, DIM:2 * DIM] * mlp_rows(rb_ref[...]) + lin_rows(eb_ref[...])
    )


def _tc_dense(g, rbf, edge_f, W1, b1, W2, b2, W3, b3):
    blk2 = pl.BlockSpec((TC_BLK2, 2 * DIM), lambda i: (i, 0))
    blk_ta = pl.BlockSpec((DIM, TC_BLK2), lambda i: (0, i))
    blk_tb = pl.BlockSpec((DIM, TC_BLK2), lambda i: (0, i + TC_NBLK))
    wspec = pl.BlockSpec((DIM, DIM), lambda i: (0, 0))
    brow = pl.BlockSpec((1, DIM), lambda i: (0, 0))
    bcol = pl.BlockSpec((DIM, 1), lambda i: (0, 0))
    return pl.pallas_call(
        _tc_body,
        grid=(TC_NBLK,),
        in_specs=[blk2, blk_ta, blk_tb, blk_ta, blk_tb,
                  wspec, bcol, wspec, brow, wspec, brow],
        out_specs=blk2,
        out_shape=jax.ShapeDtypeStruct((E_HALF, 2 * DIM), jnp.float32),
    )(g, rbf.T, rbf.T, edge_f.T, edge_f.T, W1.T,
      b1.reshape(DIM, 1), W2, b2.reshape(1, DIM), W3, b3.reshape(1, DIM))


HALF = N_NODES // NC
N_TRASH = 16
ACC_ROWS = HALF + N_TRASH + 8
S_BW = 80
S_KB = 5
S_PER_TILE = N_EDGES // NS
S_ROWS_PER_TILE = S_PER_TILE // S_BW
S_NCH = S_ROWS_PER_TILE // S_KB
S_WR = HALF // NS
S_WR_TAIL = HALF - S_WR * NS

_scatter_mesh = plsc.VectorSubcoreMesh(core_axis_name="c", subcore_axis_name="s")


@functools.partial(
    pl.kernel,
    mesh=_scatter_mesh,
    compiler_params=pltpu.CompilerParams(use_tc_tiling_on_sc=False),
    out_type=jax.ShapeDtypeStruct((N_NODES, DIM), jnp.float32),
    scratch_types=[
        pltpu.VMEM((S_KB, S_BW), jnp.int32),
        [pltpu.VMEM((S_BW, DIM), jnp.float32) for _ in range(S_KB)],
        pltpu.VMEM_SHARED((ACC_ROWS, DIM), jnp.float32),
        pltpu.SemaphoreType.DMA,
        pltpu.SemaphoreType.DMA,
    ],
)
def _sc_scatter(msg_hbm, dst2d_hbm, z_hbm, out_hbm, dbuf, mbufs, acc,
                sem_in, sem_sc):
    c = lax.axis_index("c")
    s = lax.axis_index("s")
    base = c * HALF

    @pl.when(s == 0)
    def _():
        pltpu.sync_copy(z_hbm, acc)

    plsc.subcore_barrier()

    row0 = s * S_ROWS_PER_TILE
    mrow0 = (s % 8) * S_PER_TILE

    def group(g, carry):
        rb = row0 + g * S_KB
        pltpu.sync_copy(dst2d_hbm.at[pl.ds(rb, S_KB)], dbuf)
        for j in range(S_KB):
            for t in range(S_BW // 16):
                d = dbuf[j, pl.ds(t * 16, 16)]
                ok = (d >= base) & (d < base + HALF)
                trash = HALF + (d & (N_TRASH - 1))
                dbuf[j, pl.ds(t * 16, 16)] = jnp.where(ok, d - base, trash)
        mr = mrow0 + g * S_KB * S_BW

        @pl.when(s < 8)
        def _():
            for j in range(S_KB):
                pltpu.async_copy(
                    msg_hbm.at[pl.ds(mr + j * S_BW, S_BW), pl.ds(0, DIM)],
                    mbufs[j], sem_in,
                )

        @pl.when(s >= 8)
        def _():
            for j in range(S_KB):
                pltpu.async_copy(
                    msg_hbm.at[pl.ds(mr + j * S_BW, S_BW), pl.ds(DIM, DIM)],
                    mbufs[j], sem_in,
                )

        stores = []
        for j in range(S_KB):
            pltpu.make_async_copy(
                msg_hbm.at[pl.ds(mr + j * S_BW, S_BW), pl.ds(0, DIM)],
                mbufs[j], sem_in,
            ).wait()
            stores.append(
                pltpu.async_copy(
                    mbufs[j], acc.at[dbuf.at[j]], sem_sc, add=True
                )
            )
        for cp in stores:
            cp.wait()
        return carry

    lax.fori_loop(0, S_NCH, group, None)
    plsc.subcore_barrier()

    pltpu.sync_copy(
        acc.at[pl.ds(s * S_WR, S_WR)],
        out_hbm.at[pl.ds(base + s * S_WR, S_WR)],
    )

    @pl.when(s == 0)
    def _():
        pltpu.sync_copy(
            acc.at[pl.ds(NS * S_WR, S_WR_TAIL)],
            out_hbm.at[pl.ds(base + NS * S_WR, S_WR_TAIL)],
        )


def kernel(new_node, rbf, edge_f, edge_index, W1, b1, W2, b2, W3, b3):
    src = edge_index[0].astype(jnp.int32)
    dst = edge_index[1].astype(jnp.int32)
    src2d = src.reshape(N_EDGES // G_IDXW, G_IDXW)
    dst2d = dst.reshape(N_EDGES // S_BW, S_BW)
    g = _sc_gather(new_node.astype(jnp.bfloat16), src2d)
    msg = _tc_dense(g, rbf, edge_f, W1, b1, W2, b2, W3, b3)
    z = jnp.zeros((ACC_ROWS, DIM), jnp.float32)
    return _sc_scatter(msg, dst2d, z)


# speedup vs baseline: 3.3612x; 1.0824x over previous
import functools

import jax
import jax.numpy as jnp
from jax import lax
from jax.experimental import pallas as pl
from jax.experimental.pallas import tpu as pltpu
from jax.experimental.pallas import tpu_sc as plsc

N_NODES = 50000
N_EDGES = 800000
E_HALF = N_EDGES // 2
DIM = 64

NC = 2
NS = 16
NW = NC * NS

G_IDXW = 125
G_ROWS_PER_W = (N_EDGES // G_IDXW) // NW
G_CHUNK_ROWS = 8
G_NCH = G_ROWS_PER_W // G_CHUNK_ROWS
G_CHUNK_E = G_CHUNK_ROWS * G_IDXW
G_EDGES_PER_W = N_EDGES // NW

_gather_mesh = plsc.VectorSubcoreMesh(core_axis_name="c", subcore_axis_name="s")


@functools.partial(
    pl.kernel,
    mesh=_gather_mesh,
    compiler_params=pltpu.CompilerParams(use_tc_tiling_on_sc=False),
    out_type=jax.ShapeDtypeStruct((E_HALF, 2 * DIM), jnp.bfloat16),
    scratch_types=[
        pltpu.VMEM((G_CHUNK_ROWS, G_IDXW), jnp.int32),
        pltpu.VMEM((G_CHUNK_E, DIM), jnp.bfloat16),
        pltpu.SemaphoreType.DMA,
    ],
)
def _sc_gather(node_hbm, src2d_hbm, g_hbm, idx_v, rows_v, sem):
    c = lax.axis_index("c")
    s = lax.axis_index("s")
    wid = s * NC + c
    row0 = wid * G_ROWS_PER_W
    half_row0 = (wid % 16) * G_EDGES_PER_W

    def body(i, carry):
        rb = row0 + i * G_CHUNK_ROWS
        pltpu.sync_copy(src2d_hbm.at[pl.ds(rb, G_CHUNK_ROWS)], idx_v)
        cps = [
            pltpu.async_copy(
                node_hbm.at[idx_v.at[j]],
                rows_v.at[pl.ds(j * G_IDXW, G_IDXW)],
                sem,
            )
            for j in range(G_CHUNK_ROWS)
        ]
        for cp in cps:
            cp.wait()
        orow = half_row0 + i * G_CHUNK_E

        @pl.when(wid < 16)
        def _():
            pltpu.sync_copy(
                rows_v, g_hbm.at[pl.ds(orow, G_CHUNK_E), pl.ds(0, DIM)]
            )

        @pl.when(wid >= 16)
        def _():
            pltpu.sync_copy(
                rows_v, g_hbm.at[pl.ds(orow, G_CHUNK_E), pl.ds(DIM, DIM)]
            )

        return carry

    lax.fori_loop(0, G_NCH, body, None)


TC_BLK2 = 3200
TC_NBLK = E_HALF // TC_BLK2


def _tc_body(g_ref, ra_ref, rb_ref, ea_ref, eb_ref, w1t_ref, b1_ref,
             w2_ref, b2_ref, w3_ref, b3_ref, o_ref):
    def mlp_rows(rt):
        ht = jnp.dot(w1t_ref[...], rt, preferred_element_type=jnp.float32)
        ht = ht + b1_ref[...]
        z = 0.5 * ht
        sp = 2.0 * jnp.log1p(jnp.exp(jnp.minimum(z, 14.0)))
        ht = jnp.where(z > 14.0, ht, sp)
        h = lax.dot_general(ht, w2_ref[...], (((0,), (0,)), ((), ())),
                            preferred_element_type=jnp.float32)
        return h + b2_ref[...]

    def lin_rows(et):
        e2 = lax.dot_general(et, w3_ref[...], (((0,), (0,)), ((), ())),
                             preferred_element_type=jnp.float32)
        return e2 + b3_ref[...]

    g = g_ref[...].astype(jnp.float32)
    o_ref[:, 0:DIM] = g[:, 0:DIM] * mlp_rows(ra_ref[...]) + lin_rows(ea_ref[...])
    o_ref[:, DIM:2 * DIM] = (
        g[:, DIM:2 * DIM] * mlp_rows(rb_ref[...]) + lin_rows(eb_ref[...])
    )


def _tc_dense_part(blk_lo, nblk):
    blk2 = pl.BlockSpec((TC_BLK2, 2 * DIM), lambda i: (i + blk_lo, 0))
    blk2o = pl.BlockSpec((TC_BLK2, 2 * DIM), lambda i: (i, 0))
    blk_ta = pl.BlockSpec((DIM, TC_BLK2), lambda i: (0, i + blk_lo))
    blk_tb = pl.BlockSpec((DIM, TC_BLK2), lambda i: (0, i + blk_lo + TC_NBLK))
    wspec = pl.BlockSpec((DIM, DIM), lambda i: (0, 0))
    brow = pl.BlockSpec((1, DIM), lambda i: (0, 0))
    bcol = pl.BlockSpec((DIM, 1), lambda i: (0, 0))

    def run(g, rbf, edge_f, W1, b1, W2, b2, W3, b3):
        return pl.pallas_call(
            _tc_body,
            grid=(nblk,),
            in_specs=[blk2, blk_ta, blk_tb, blk_ta, blk_tb,
                      wspec, bcol, wspec, brow, wspec, brow],
            out_specs=blk2o,
            out_shape=jax.ShapeDtypeStruct((nblk * TC_BLK2, 2 * DIM),
                                           jnp.float32),
        )(g, rbf.T, rbf.T, edge_f.T, edge_f.T, W1.T,
          b1.reshape(DIM, 1), W2, b2.reshape(1, DIM), W3, b3.reshape(1, DIM))

    return run


HALF = N_NODES // NC
N_TRASH = 16
ACC_ROWS = HALF + N_TRASH + 8
S_BW = 80
S_KB = 5
S_GROUP = S_KB * S_BW
S_WR = HALF // NS
S_WR_TAIL = HALF - S_WR * NS

_scatter_mesh = plsc.VectorSubcoreMesh(core_axis_name="c", subcore_axis_name="s")


def _make_scatter(paired_lo, n_paired):
    per_tile = n_paired // 8
    groups = per_tile // S_GROUP
    assert groups * S_GROUP == per_tile

    @functools.partial(
        pl.kernel,
        mesh=_scatter_mesh,
        compiler_params=pltpu.CompilerParams(use_tc_tiling_on_sc=False),
        out_type=jax.ShapeDtypeStruct((N_NODES, DIM), jnp.float32),
        scratch_types=[
            pltpu.VMEM((S_KB, S_BW), jnp.int32),
            [pltpu.VMEM((S_BW, DIM), jnp.float32) for _ in range(S_KB)],
            pltpu.VMEM_SHARED((ACC_ROWS, DIM), jnp.float32),
            pltpu.SemaphoreType.DMA,
            pltpu.SemaphoreType.DMA,
        ],
    )
    def scatter(msg_hbm, dst2d_hbm, z_hbm, out_hbm, dbuf, mbufs, acc,
                sem_in, sem_sc):
        c = lax.axis_index("c")
        s = lax.axis_index("s")
        base = c * HALF

        @pl.when(s == 0)
        def _():
            pltpu.sync_copy(z_hbm, acc)

        plsc.subcore_barrier()

        side = s // 8
        e0 = paired_lo + (s % 8) * per_tile + side * E_HALF
        drow0 = e0 // S_BW
        mrow0 = (s % 8) * per_tile

        def group(g, carry):
            pltpu.sync_copy(dst2d_hbm.at[pl.ds(drow0 + g * S_KB, S_KB)], dbuf)
            for j in range(S_KB):
                for t in range(S_BW // 16):
                    d = dbuf[j, pl.ds(t * 16, 16)]
                    ok = (d >= base) & (d < base + HALF)
                    trash = HALF + (d & (N_TRASH - 1))
                    dbuf[j, pl.ds(t * 16, 16)] = jnp.where(ok, d - base, trash)
            mr = mrow0 + g * S_GROUP

            @pl.when(side == 0)
            def _():
                for j in range(S_KB):
                    pltpu.async_copy(
                        msg_hbm.at[pl.ds(mr + j * S_BW, S_BW), pl.ds(0, DIM)],
                        mbufs[j], sem_in,
                    )

            @pl.when(side == 1)
            def _():
                for j in range(S_KB):
                    pltpu.async_copy(
                        msg_hbm.at[pl.ds(mr + j * S_BW, S_BW),
                                   pl.ds(DIM, DIM)],
                        mbufs[j], sem_in,
                    )

            stores = []
            for j in range(S_KB):
                pltpu.make_async_copy(
                    msg_hbm.at[pl.ds(mr + j * S_BW, S_BW), pl.ds(0, DIM)],
                    mbufs[j], sem_in,
                ).wait()
                stores.append(
                    pltpu.async_copy(
                        mbufs[j], acc.at[dbuf.at[j]], sem_sc, add=True
                    )
                )
            for cp in stores:
                cp.wait()
            return carry

        lax.fori_loop(0, groups, group, None)
        plsc.subcore_barrier()

        pltpu.sync_copy(
            acc.at[pl.ds(s * S_WR, S_WR)],
            out_hbm.at[pl.ds(base + s * S_WR, S_WR)],
        )

        @pl.when(s == 0)
        def _():
            pltpu.sync_copy(
                acc.at[pl.ds(NS * S_WR, S_WR_TAIL)],
                out_hbm.at[pl.ds(base + NS * S_WR, S_WR_TAIL)],
            )

    return scatter


P1_BLKS = 62
P1_ROWS = P1_BLKS * TC_BLK2
P2_BLKS = TC_NBLK - P1_BLKS
P2_ROWS = P2_BLKS * TC_BLK2

_dense_p1 = _tc_dense_part(0, P1_BLKS)
_dense_p2 = _tc_dense_part(P1_BLKS, P2_BLKS)
_scatter_p1 = _make_scatter(0, P1_ROWS)
_scatter_p2 = _make_scatter(P1_ROWS, P2_ROWS)


def kernel(new_node, rbf, edge_f, edge_index, W1, b1, W2, b2, W3, b3):
    src = edge_index[0].astype(jnp.int32)
    dst = edge_index[1].astype(jnp.int32)
    src2d = src.reshape(N_EDGES // G_IDXW, G_IDXW)
    dst2d = dst.reshape(N_EDGES // S_BW, S_BW)
    g = _sc_gather(new_node.astype(jnp.bfloat16), src2d)
    z = jnp.zeros((ACC_ROWS, DIM), jnp.float32)
    m1 = _dense_p1(g, rbf, edge_f, W1, b1, W2, b2, W3, b3)
    o1 = _scatter_p1(m1, dst2d, z)
    m2 = _dense_p2(g, rbf, edge_f, W1, b1, W2, b2, W3, b3)
    o2 = _scatter_p2(m2, dst2d, z)
    return o1 + o2


# speedup vs baseline: 3.5975x; 1.0703x over previous
import functools

import jax
import jax.numpy as jnp
from jax import lax
from jax.experimental import pallas as pl
from jax.experimental.pallas import tpu as pltpu
from jax.experimental.pallas import tpu_sc as plsc

N_NODES = 50000
N_EDGES = 800000
E_HALF = N_EDGES // 2
DIM = 64

NC = 2
NS = 16
NW = NC * NS

G_IDXW = 125
G_CHUNK_ROWS = 8
G_CHUNK_E = G_CHUNK_ROWS * G_IDXW

_gather_mesh = plsc.VectorSubcoreMesh(core_axis_name="c", subcore_axis_name="s")


def _make_gather(P0, n_rows):
    n_chunks = n_rows // G_CHUNK_E
    assert n_chunks * G_CHUNK_E == n_rows and P0 % (8 * G_IDXW) == 0
    n_base, n_rem = divmod(n_chunks, NW)

    @functools.partial(
        pl.kernel,
        mesh=_gather_mesh,
        compiler_params=pltpu.CompilerParams(use_tc_tiling_on_sc=False),
        out_type=jax.ShapeDtypeStruct((n_rows, 2 * DIM), jnp.bfloat16),
        scratch_types=[
            pltpu.VMEM((G_CHUNK_ROWS, G_IDXW), jnp.int32),
            pltpu.VMEM((G_CHUNK_E, DIM), jnp.bfloat16),
            pltpu.SemaphoreType.DMA,
        ],
    )
    def gather(node_hbm, src2d_hbm, g_hbm, idx_v, rows_v, sem):
        c = lax.axis_index("c")
        s = lax.axis_index("s")
        wid = s * NC + c
        if n_rem == 0:
            ntrip = n_base
        else:
            ntrip = n_base + (wid < n_rem).astype(jnp.int32)

        def body(i, carry):
            ch = wid + i * NW
            for side in range(2):
                srow = (side * E_HALF + P0) // G_IDXW + ch * G_CHUNK_ROWS
                pltpu.sync_copy(src2d_hbm.at[pl.ds(srow, G_CHUNK_ROWS)], idx_v)
                cps = [
                    pltpu.async_copy(
                        node_hbm.at[idx_v.at[j]],
                        rows_v.at[pl.ds(j * G_IDXW, G_IDXW)],
                        sem,
                    )
                    for j in range(G_CHUNK_ROWS)
                ]
                for cp in cps:
                    cp.wait()
                pltpu.sync_copy(
                    rows_v,
                    g_hbm.at[pl.ds(ch * G_CHUNK_E, G_CHUNK_E),
                             pl.ds(side * DIM, DIM)],
                )
            return carry

        lax.fori_loop(0, ntrip, body, None)

    return gather


TC_BLK2 = 3200
TC_NBLK = E_HALF // TC_BLK2


def _tc_body(g_ref, ra_ref, rb_ref, ea_ref, eb_ref, w1t_ref, b1_ref,
             w2_ref, b2_ref, w3_ref, b3_ref, o_ref):
    def mlp_rows(rt):
        ht = jnp.dot(w1t_ref[...], rt, preferred_element_type=jnp.float32)
        ht = ht + b1_ref[...]
        z = 0.5 * ht
        sp = 2.0 * jnp.log1p(jnp.exp(jnp.minimum(z, 14.0)))
        ht = jnp.where(z > 14.0, ht, sp)
        h = lax.dot_general(ht, w2_ref[...], (((0,), (0,)), ((), ())),
                            preferred_element_type=jnp.float32)
        return h + b2_ref[...]

    def lin_rows(et):
        e2 = lax.dot_general(et, w3_ref[...], (((0,), (0,)), ((), ())),
                             preferred_element_type=jnp.float32)
        return e2 + b3_ref[...]

    g = g_ref[...].astype(jnp.float32)
    o_ref[:, 0:DIM] = g[:, 0:DIM] * mlp_rows(ra_ref[...]) + lin_rows(ea_ref[...])
    o_ref[:, DIM:2 * DIM] = (
        g[:, DIM:2 * DIM] * mlp_rows(rb_ref[...]) + lin_rows(eb_ref[...])
    )


def _tc_dense_part(blk_lo, nblk):
    blk2 = pl.BlockSpec((TC_BLK2, 2 * DIM), lambda i: (i, 0))
    blk2o = pl.BlockSpec((TC_BLK2, 2 * DIM), lambda i: (i, 0))
    blk_ta = pl.BlockSpec((DIM, TC_BLK2), lambda i: (0, i + blk_lo))
    blk_tb = pl.BlockSpec((DIM, TC_BLK2), lambda i: (0, i + blk_lo + TC_NBLK))
    wspec = pl.BlockSpec((DIM, DIM), lambda i: (0, 0))
    brow = pl.BlockSpec((1, DIM), lambda i: (0, 0))
    bcol = pl.BlockSpec((DIM, 1), lambda i: (0, 0))

    def run(g, rbf, edge_f, W1, b1, W2, b2, W3, b3):
        return pl.pallas_call(
            _tc_body,
            grid=(nblk,),
            in_specs=[blk2, blk_ta, blk_tb, blk_ta, blk_tb,
                      wspec, bcol, wspec, brow, wspec, brow],
            out_specs=blk2o,
            out_shape=jax.ShapeDtypeStruct((nblk * TC_BLK2, 2 * DIM),
                                           jnp.float32),
        )(g, rbf.T, rbf.T, edge_f.T, edge_f.T, W1.T,
          b1.reshape(DIM, 1), W2, b2.reshape(1, DIM), W3, b3.reshape(1, DIM))

    return run


HALF = N_NODES // NC
N_TRASH = 16
ACC_ROWS = HALF + N_TRASH + 8
S_BW = 80
S_KB = 5
S_GROUP = S_KB * S_BW
S_WR = HALF // NS
S_WR_TAIL = HALF - S_WR * NS

_scatter_mesh = plsc.VectorSubcoreMesh(core_axis_name="c", subcore_axis_name="s")


def _make_scatter(paired_lo, n_paired):
    per_tile = n_paired // 8
    groups = per_tile // S_GROUP
    assert groups * S_GROUP == per_tile

    @functools.partial(
        pl.kernel,
        mesh=_scatter_mesh,
        compiler_params=pltpu.CompilerParams(use_tc_tiling_on_sc=False),
        out_type=jax.ShapeDtypeStruct((N_NODES, DIM), jnp.float32),
        scratch_types=[
            pltpu.VMEM((S_KB, S_BW), jnp.int32),
            [pltpu.VMEM((S_BW, DIM), jnp.float32) for _ in range(S_KB)],
            pltpu.VMEM_SHARED((ACC_ROWS, DIM), jnp.float32),
            pltpu.SemaphoreType.DMA,
            pltpu.SemaphoreType.DMA,
        ],
    )
    def scatter(msg_hbm, dst2d_hbm, z_hbm, out_hbm, dbuf, mbufs, acc,
                sem_in, sem_sc):
        c = lax.axis_index("c")
        s = lax.axis_index("s")
        base = c * HALF

        @pl.when(s == 0)
        def _():
            pltpu.sync_copy(z_hbm, acc)

        plsc.subcore_barrier()

        side = s // 8
        e0 = paired_lo + (s % 8) * per_tile + side * E_HALF
        drow0 = e0 // S_BW
        mrow0 = (s % 8) * per_tile

        def group(g, carry):
            pltpu.sync_copy(dst2d_hbm.at[pl.ds(drow0 + g * S_KB, S_KB)], dbuf)
            for j in range(S_KB):
                for t in range(S_BW // 16):
                    d = dbuf[j, pl.ds(t * 16, 16)]
                    ok = (d >= base) & (d < base + HALF)
                    trash = HALF + (d & (N_TRASH - 1))
                    dbuf[j, pl.ds(t * 16, 16)] = jnp.where(ok, d - base, trash)
            mr = mrow0 + g * S_GROUP

            @pl.when(side == 0)
            def _():
                for j in range(S_KB):
                    pltpu.async_copy(
                        msg_hbm.at[pl.ds(mr + j * S_BW, S_BW), pl.ds(0, DIM)],
                        mbufs[j], sem_in,
                    )

            @pl.when(side == 1)
            def _():
                for j in range(S_KB):
                    pltpu.async_copy(
                        msg_hbm.at[pl.ds(mr + j * S_BW, S_BW),
                                   pl.ds(DIM, DIM)],
                        mbufs[j], sem_in,
                    )

            stores = []
            for j in range(S_KB):
                pltpu.make_async_copy(
                    msg_hbm.at[pl.ds(mr + j * S_BW, S_BW), pl.ds(0, DIM)],
                    mbufs[j], sem_in,
                ).wait()
                stores.append(
                    pltpu.async_copy(
                        mbufs[j], acc.at[dbuf.at[j]], sem_sc, add=True
                    )
                )
            for cp in stores:
                cp.wait()
            return carry

        lax.fori_loop(0, groups, group, None)
        plsc.subcore_barrier()

        pltpu.sync_copy(
            acc.at[pl.ds(s * S_WR, S_WR)],
            out_hbm.at[pl.ds(base + s * S_WR, S_WR)],
        )

        @pl.when(s == 0)
        def _():
            pltpu.sync_copy(
                acc.at[pl.ds(NS * S_WR, S_WR_TAIL)],
                out_hbm.at[pl.ds(base + NS * S_WR, S_WR_TAIL)],
            )

    return scatter


_PARTS = (50, 50, 25)
_part_lo = [sum(_PARTS[:k]) for k in range(len(_PARTS))]
_gathers = [_make_gather(lo * TC_BLK2, n * TC_BLK2)
            for lo, n in zip(_part_lo, _PARTS)]
_denses = [_tc_dense_part(lo, n) for lo, n in zip(_part_lo, _PARTS)]
_scatters = [_make_scatter(lo * TC_BLK2, n * TC_BLK2)
             for lo, n in zip(_part_lo, _PARTS)]


def kernel(new_node, rbf, edge_f, edge_index, W1, b1, W2, b2, W3, b3):
    src = edge_index[0].astype(jnp.int32)
    dst = edge_index[1].astype(jnp.int32)
    src2d = src.reshape(N_EDGES // G_IDXW, G_IDXW)
    dst2d = dst.reshape(N_EDGES // S_BW, S_BW)
    node_bf = new_node.astype(jnp.bfloat16)
    z = jnp.zeros((ACC_ROWS, DIM), jnp.float32)
    gs = [gat(node_bf, src2d) for gat in _gathers]
    outs = []
    for k in range(len(_PARTS)):
        m = _denses[k](gs[k], rbf, edge_f, W1, b1, W2, b2, W3, b3)
        outs.append(_scatters[k](m, dst2d, z))
    return outs[0] + outs[1] + outs[2]
